# Initial kernel scaffold; baseline (speedup 1.0000x reference)
import jax
import jax.numpy as jnp
from jax.experimental import pallas as pl


def kernel(z_drug, z_adr, edge_label_index, type):
    raise NotImplementedError("write your pallas kernel here")


# speedup vs baseline: 1.1438x; 1.1438x over previous
import functools

import jax
import jax.numpy as jnp
from jax import lax
from jax.experimental import pallas as pl
from jax.experimental.pallas import tpu as pltpu
from jax.experimental.pallas import tpu_sc as plsc

_EPS = 1e-6
_NC = 2
_NS = 16
_LANES = 16


def _normalize_tables(z_drug, z_adr):
    def body(zd_ref, za_ref, od_ref, oa_ref):
        for ref, out in ((zd_ref, od_ref), (za_ref, oa_ref)):
            x = ref[...]
            ss = jnp.sum(x * x, axis=1, keepdims=True)
            n = jnp.maximum(jnp.sqrt(ss), _EPS)
            out[...] = x * (1.0 / n)

    return pl.pallas_call(
        body,
        out_shape=(
            jax.ShapeDtypeStruct(z_drug.shape, jnp.float32),
            jax.ShapeDtypeStruct(z_adr.shape, jnp.float32),
        ),
    )(z_drug, z_adr)


@functools.lru_cache(maxsize=None)
def _make_sc_kernel(E, D, C):
    NW = _NC * _NS
    per_w = E // NW
    n_chunks = per_w // C
    mesh = plsc.VectorSubcoreMesh(core_axis_name="c", subcore_axis_name="s")

    @functools.partial(
        pl.kernel,
        mesh=mesh,
        compiler_params=pltpu.CompilerParams(needs_layout_passes=False),
        out_type=jax.ShapeDtypeStruct((E,), jnp.float32),
        scratch_types=[
            pltpu.VMEM((C,), jnp.int32),
            pltpu.VMEM((C,), jnp.int32),
            pltpu.VMEM((C, D), jnp.float32),
            pltpu.VMEM((C, D), jnp.float32),
            pltpu.VMEM((C,), jnp.float32),
            pltpu.SemaphoreType.DMA,
        ],
    )
    def sc_k(zd_hbm, za_hbm, i0_hbm, i1_hbm, out_hbm, i0_v, i1_v, a_v, b_v, o_v, sem):
        wid = lax.axis_index("s") * _NC + lax.axis_index("c")
        base = wid * per_w

        def chunk_body(c, carry):
            off = base + c * C
            pltpu.sync_copy(i0_hbm.at[pl.ds(off, C)], i0_v)
            pltpu.sync_copy(i1_hbm.at[pl.ds(off, C)], i1_v)
            cp0 = pltpu.async_copy(zd_hbm.at[i0_v], a_v, sem)
            cp1 = pltpu.async_copy(za_hbm.at[i1_v], b_v, sem)
            cp0.wait()
            cp1.wait()

            lane = lax.iota(jnp.int32, _LANES)

            def group_body(g, gcarry):
                row = lane + g * _LANES
                col0 = jnp.zeros((_LANES,), jnp.int32)
                acc = plsc.load_gather(a_v, [row, col0]) * plsc.load_gather(
                    b_v, [row, col0]
                )
                for j in range(1, D):
                    col = jnp.full((_LANES,), j, jnp.int32)
                    va = plsc.load_gather(a_v, [row, col])
                    vb = plsc.load_gather(b_v, [row, col])
                    acc = acc + va * vb
                o_v[pl.ds(g * _LANES, _LANES)] = acc
                return gcarry

            lax.fori_loop(0, C // _LANES, group_body, 0)
            pltpu.sync_copy(o_v, out_hbm.at[pl.ds(off, C)])
            return carry

        lax.fori_loop(0, n_chunks, chunk_body, 0)

    return sc_k


def kernel(z_drug, z_adr, edge_label_index, type):
    row0 = edge_label_index[0].astype(jnp.int32)
    row1 = edge_label_index[1].astype(jnp.int32)
    zd, za = _normalize_tables(
        z_drug.astype(jnp.float32), z_adr.astype(jnp.float32)
    )
    E = row0.shape[0]
    D = z_drug.shape[1]
    C = 80
    sck = _make_sc_kernel(E, D, C)
    return sck(zd, za, row0, row1)


# speedup vs baseline: 1.3835x; 1.2095x over previous
import functools

import jax
import jax.numpy as jnp
from jax import lax
from jax.experimental import pallas as pl
from jax.experimental.pallas import tpu as pltpu
from jax.experimental.pallas import tpu_sc as plsc

_EPS = 1e-6
_NC = 2
_NS = 16
_LANES = 16
_CH = 128


def _normalize_tables(z_drug, z_adr):
    def body(zd_ref, za_ref, od_ref, oa_ref):
        for ref, out in ((zd_ref, od_ref), (za_ref, oa_ref)):
            x = ref[...]
            ss = jnp.sum(x * x, axis=1, keepdims=True)
            n = jnp.maximum(jnp.sqrt(ss), _EPS)
            out[...] = x * (1.0 / n)

    return pl.pallas_call(
        body,
        out_shape=(
            jax.ShapeDtypeStruct(z_drug.shape, jnp.float32),
            jax.ShapeDtypeStruct(z_adr.shape, jnp.float32),
        ),
    )(z_drug, z_adr)


@functools.lru_cache(maxsize=None)
def _make_sc_kernel(E_pad, D, K):
    NW = _NC * _NS
    n_slots = K * NW
    mesh = plsc.VectorSubcoreMesh(core_axis_name="c", subcore_axis_name="s")

    @functools.partial(
        pl.kernel,
        mesh=mesh,
        compiler_params=pltpu.CompilerParams(needs_layout_passes=False),
        out_type=jax.ShapeDtypeStruct((E_pad,), jnp.float32),
        scratch_types=[
            pltpu.VMEM((K, 2, _CH), jnp.int32),
            pltpu.VMEM((2, _CH, D), jnp.float32),
            pltpu.VMEM((2, _CH, D), jnp.float32),
            pltpu.VMEM((2, _CH), jnp.float32),
            pltpu.SemaphoreType.DMA,
            pltpu.SemaphoreType.DMA,
            pltpu.SemaphoreType.DMA,
            pltpu.SemaphoreType.DMA,
        ],
    )
    def sc_k(zd_hbm, za_hbm, idx_hbm, out_hbm, idx_all, a_ring, b_ring,
             o_ring, sg0, sg1, so0, so1):
        wid = lax.axis_index("s") * _NC + lax.axis_index("c")
        base_chunk = wid * K
        pltpu.sync_copy(idx_hbm.at[pl.ds(base_chunk, K)], idx_all)

        sgs = (sg0, sg1)
        sos = (so0, so1)

        def fire_gather(k, r):
            pltpu.async_copy(zd_hbm.at[idx_all.at[k, 0]], a_ring.at[r], sgs[r])
            pltpu.async_copy(za_hbm.at[idx_all.at[k, 1]], b_ring.at[r], sgs[r])

        def wait_gather(r):
            pltpu.make_async_copy(
                zd_hbm.at[idx_all.at[0, 0]], a_ring.at[r], sgs[r]).wait()
            pltpu.make_async_copy(
                za_hbm.at[idx_all.at[0, 1]], b_ring.at[r], sgs[r]).wait()

        def wait_store(r):
            pltpu.make_async_copy(
                o_ring.at[r], out_hbm.at[pl.ds(0, _CH)], sos[r]).wait()

        fire_gather(0, 0)
        fire_gather(1, 1)

        lane = lax.iota(jnp.int32, _LANES)
        cols = [jnp.full((_LANES,), j, jnp.int32) for j in range(D)]

        def step(t, carry):
            for r in range(2):
                k = 2 * t + r
                wait_gather(r)

                @pl.when(k >= 2)
                def _():
                    wait_store(r)

                a_v = a_ring.at[r]
                b_v = b_ring.at[r]

                def group(g, gc):
                    row = lane + g * _LANES
                    acc = plsc.load_gather(a_v, [row, cols[0]]) * \
                        plsc.load_gather(b_v, [row, cols[0]])
                    for j in range(1, D):
                        va = plsc.load_gather(a_v, [row, cols[j]])
                        vb = plsc.load_gather(b_v, [row, cols[j]])
                        acc = acc + va * vb
                    o_ring[r, pl.ds(g * _LANES, _LANES)] = acc
                    return gc

                lax.fori_loop(0, _CH // _LANES, group, 0)
                pltpu.async_copy(
                    o_ring.at[r],
                    out_hbm.at[pl.ds((base_chunk + k) * _CH, _CH)],
                    sos[r],
                )

                @pl.when(k + 2 < K)
                def _():
                    fire_gather(k + 2, r)
            return carry

        lax.fori_loop(0, K // 2, step, 0)
        wait_store(0)
        wait_store(1)

    return sc_k


def kernel(z_drug, z_adr, edge_label_index, type):
    zd, za = _normalize_tables(
        z_drug.astype(jnp.float32), z_adr.astype(jnp.float32)
    )
    E = edge_label_index.shape[1]
    D = z_drug.shape[1]
    NW = _NC * _NS
    n_chunks = -(-E // _CH)
    K = -(-n_chunks // NW)
    K += K % 2
    E_pad = K * NW * _CH

    idx = edge_label_index.astype(jnp.int32)
    idx = jnp.pad(idx, ((0, 0), (0, E_pad - E)))
    idx = idx.reshape(2, K * NW, _CH).transpose(1, 0, 2)

    sck = _make_sc_kernel(E_pad, D, K)
    sim = sck(zd, za, idx)
    return sim[:E]


# speedup vs baseline: 2.8510x; 2.0607x over previous
import functools

import jax
import jax.numpy as jnp
from jax import lax
from jax.experimental import pallas as pl
from jax.experimental.pallas import tpu as pltpu
from jax.experimental.pallas import tpu_sc as plsc

_EPS = 1e-6
_NC = 2
_NS = 16
_LANES = 16
_CH = 128


def _normalize_tables(z_drug, z_adr):
    def body(zd_ref, za_ref, od_ref, oa_ref):
        for ref, out in ((zd_ref, od_ref), (za_ref, oa_ref)):
            x = ref[...]
            ss = jnp.sum(x * x, axis=1, keepdims=True)
            n = jnp.maximum(jnp.sqrt(ss), _EPS)
            out[...] = x * (1.0 / n)

    return pl.pallas_call(
        body,
        out_shape=(
            jax.ShapeDtypeStruct(z_drug.shape, jnp.float32),
            jax.ShapeDtypeStruct(z_adr.shape, jnp.float32),
        ),
    )(z_drug, z_adr)


@functools.lru_cache(maxsize=None)
def _make_sc_kernel(E_pad, D, K):
    NW = _NC * _NS
    n_slots = K * NW
    mesh = plsc.VectorSubcoreMesh(core_axis_name="c", subcore_axis_name="s")

    @functools.partial(
        pl.kernel,
        mesh=mesh,
        compiler_params=pltpu.CompilerParams(needs_layout_passes=False),
        out_type=jax.ShapeDtypeStruct((E_pad,), jnp.float32),
        scratch_types=[
            pltpu.VMEM((K, 2, _CH), jnp.int32),
            pltpu.VMEM((2, _CH, D), jnp.float32),
            pltpu.VMEM((2, _CH, D), jnp.float32),
            pltpu.VMEM((2, _CH), jnp.float32),
            pltpu.SemaphoreType.DMA,
            pltpu.SemaphoreType.DMA,
            pltpu.SemaphoreType.DMA,
            pltpu.SemaphoreType.DMA,
        ],
    )
    def sc_k(zd_hbm, za_hbm, idx_hbm, out_hbm, idx_all, a_ring, b_ring,
             o_ring, sg0, sg1, so0, so1):
        wid = lax.axis_index("s") * _NC + lax.axis_index("c")
        base_chunk = wid * K
        pltpu.sync_copy(idx_hbm.at[pl.ds(base_chunk, K)], idx_all)

        sgs = (sg0, sg1)
        sos = (so0, so1)

        def fire_gather(k, r):
            pltpu.async_copy(zd_hbm.at[idx_all.at[k, 0]], a_ring.at[r], sgs[r])
            pltpu.async_copy(za_hbm.at[idx_all.at[k, 1]], b_ring.at[r], sgs[r])

        def wait_gather(r):
            pltpu.make_async_copy(
                zd_hbm.at[idx_all.at[0, 0]], a_ring.at[r], sgs[r]).wait()
            pltpu.make_async_copy(
                za_hbm.at[idx_all.at[0, 1]], b_ring.at[r], sgs[r]).wait()

        def wait_store(r):
            pltpu.make_async_copy(
                o_ring.at[r], out_hbm.at[pl.ds(0, _CH)], sos[r]).wait()

        fire_gather(0, 0)
        fire_gather(1, 1)

        lane = lax.iota(jnp.int32, _LANES)

        def step(t, carry):
            for r in range(2):
                k = 2 * t + r
                wait_gather(r)

                @pl.when(k >= 2)
                def _():
                    wait_store(r)

                def group(g, gc):
                    ovec = jnp.zeros((_LANES,), jnp.float32)
                    for l in range(_LANES):
                        i = g * _LANES + l
                        acc = a_ring[r, i, pl.ds(0, _LANES)] * \
                            b_ring[r, i, pl.ds(0, _LANES)]
                        for j in range(1, D // _LANES):
                            acc = acc + \
                                a_ring[r, i, pl.ds(j * _LANES, _LANES)] * \
                                b_ring[r, i, pl.ds(j * _LANES, _LANES)]
                        ovec = jnp.where(lane == l, jnp.sum(acc), ovec)
                    o_ring[r, pl.ds(g * _LANES, _LANES)] = ovec
                    return gc

                lax.fori_loop(0, _CH // _LANES, group, 0)
                pltpu.async_copy(
                    o_ring.at[r],
                    out_hbm.at[pl.ds((base_chunk + k) * _CH, _CH)],
                    sos[r],
                )

                @pl.when(k + 2 < K)
                def _():
                    fire_gather(k + 2, r)
            return carry

        lax.fori_loop(0, K // 2, step, 0)
        wait_store(0)
        wait_store(1)

    return sc_k


def kernel(z_drug, z_adr, edge_label_index, type):
    zd, za = _normalize_tables(
        z_drug.astype(jnp.float32), z_adr.astype(jnp.float32)
    )
    E = edge_label_index.shape[1]
    D = z_drug.shape[1]
    NW = _NC * _NS
    n_chunks = -(-E // _CH)
    K = -(-n_chunks // NW)
    K += K % 2
    E_pad = K * NW * _CH

    idx = edge_label_index.astype(jnp.int32)
    idx = jnp.pad(idx, ((0, 0), (0, E_pad - E)))
    idx = idx.reshape(2, K * NW, _CH).transpose(1, 0, 2)

    sck = _make_sc_kernel(E_pad, D, K)
    sim = sck(zd, za, idx)
    return sim[:E]
